# Initial kernel scaffold; baseline (speedup 1.0000x reference)
import jax
import jax.numpy as jnp
from jax.experimental import pallas as pl


def kernel(x, edge_index, batch, bn1_g, bn1_b, W1, b1, bn2_g, bn2_b, W2, b2, bn3_g, bn3_b, W3, b3, lin1_W, lin1_b, lin2_W, lin2_b):
    raise NotImplementedError("write your pallas kernel here")


# speedup vs baseline: 1.1502x; 1.1502x over previous
import functools

import jax
import jax.numpy as jnp
from jax.experimental import pallas as pl
from jax.experimental.pallas import tpu as pltpu

_BR = 1024


def _mm_body(h_ref, w_ref, o_ref, *, relu_in, ones_col):
    hb = h_ref[...]
    if relu_in:
        hb = jnp.maximum(hb, 0.0)
    col = jax.lax.broadcasted_iota(jnp.int32, hb.shape, 1)
    hb = jnp.where(col == ones_col, 1.0, hb)
    o_ref[...] = jnp.dot(hb, w_ref[...], preferred_element_type=jnp.float32)


def _affine_mm(h, w_aug, relu_in):
    n, finp = h.shape
    foutp = w_aug.shape[1]
    nb = -(-n // _BR)
    npad = nb * _BR
    if npad != n:
        h = jnp.pad(h, ((0, npad - n), (0, 0)))
    body = functools.partial(_mm_body, relu_in=relu_in, ones_col=finp - 1)
    out = pl.pallas_call(
        body,
        grid=(nb,),
        in_specs=[
            pl.BlockSpec((_BR, finp), lambda i: (i, 0)),
            pl.BlockSpec((finp, foutp), lambda i: (0, 0)),
        ],
        out_specs=pl.BlockSpec((_BR, foutp), lambda i: (i, 0)),
        out_shape=jax.ShapeDtypeStruct((npad, foutp), jnp.float32),
    )(h, w_aug)
    return out[:n]


def _pool_head_body(h_ref, bat_ref, w1_ref, w2_ref, o_ref, acc_ref, *, nb, g):
    i = pl.program_id(0)

    @pl.when(i == 0)
    def _init():
        acc_ref[...] = jnp.zeros_like(acc_ref)

    hb = jnp.maximum(h_ref[...], 0.0)
    col = jax.lax.broadcasted_iota(jnp.int32, hb.shape, 1)
    fp = hb.shape[1]
    hb = jnp.where(col == fp - 1, 1.0, hb)
    seg = bat_ref[0]
    oh = (jax.lax.broadcasted_iota(jnp.int32, (g, seg.shape[1]), 0) == seg)
    acc_ref[...] += jax.lax.dot(
        oh.astype(jnp.float32), hb, preferred_element_type=jnp.float32
    )

    @pl.when(i == nb - 1)
    def _finish():
        acc = acc_ref[...]
        cnt = acc[:, fp - 1 :]
        pooled = acc / jnp.maximum(cnt, 1.0)
        pcol = jax.lax.broadcasted_iota(jnp.int32, pooled.shape, 1)
        pooled = jnp.where(pcol == fp - 1, 1.0, pooled)
        t = jax.lax.dot(pooled, w1_ref[...], preferred_element_type=jnp.float32)
        tcol = jax.lax.broadcasted_iota(jnp.int32, t.shape, 1)
        t = jnp.where(tcol == t.shape[1] - 1, 1.0, t)
        o_ref[...] = jax.lax.dot(
            t, w2_ref[...], preferred_element_type=jnp.float32
        )


def _pool_head(h, batch, w1_aug, w2_aug, g):
    n, fp = h.shape
    nb = -(-n // _BR)
    npad = nb * _BR
    if npad != n:
        h = jnp.pad(h, ((0, npad - n), (0, 0)))
        batch = jnp.pad(batch, (0, npad - n), constant_values=2**30)
    bat3 = batch.reshape(nb, 1, _BR)
    body = functools.partial(_pool_head_body, nb=nb, g=g)
    out = pl.pallas_call(
        body,
        grid=(nb,),
        in_specs=[
            pl.BlockSpec((_BR, fp), lambda i: (i, 0)),
            pl.BlockSpec((1, 1, _BR), lambda i: (i, 0, 0)),
            pl.BlockSpec(w1_aug.shape, lambda i: (0, 0)),
            pl.BlockSpec(w2_aug.shape, lambda i: (0, 0)),
        ],
        out_specs=pl.BlockSpec((g, w2_aug.shape[1]), lambda i: (0, 0)),
        out_shape=jax.ShapeDtypeStruct((g, w2_aug.shape[1]), jnp.float32),
        scratch_shapes=[pltpu.VMEM((g, fp), jnp.float32)],
    )(h, bat3, w1_aug, w2_aug)
    return out


def _fold_bn(mean, var, gamma, beta, w):
    scale = gamma * jax.lax.rsqrt(var + 1e-5)
    bias = beta - mean * scale
    return scale[:, None] * w, bias @ w


def _pack_w(wp, brow, finp, foutp):
    fin, fout = wp.shape
    out = jnp.zeros((finp, foutp), jnp.float32)
    out = out.at[:fin, :fout].set(wp)
    out = out.at[finp - 1, :fout].set(brow)
    return out


def kernel(x, edge_index, batch, bn1_g, bn1_b, W1, b1, bn2_g, bn2_b, W2, b2,
           bn3_g, bn3_b, W3, b3, lin1_W, lin1_b, lin2_W, lin2_b):
    n = x.shape[0]
    g = 128
    src = edge_index[0]
    dst = edge_index[1]

    deg = jnp.ones((n,), jnp.float32).at[dst].add(1.0)
    dis = jax.lax.rsqrt(deg)
    norm = dis[src] * dis[dst]
    self_norm = (dis * dis)[:, None]

    def sparse_conv(xw, b):
        msg = xw[src] * norm[:, None]
        out = jnp.zeros_like(xw).at[dst].add(msg)
        return out + xw * self_norm + b

    w1p, b1row = _fold_bn(x.mean(0), x.var(0), bn1_g, bn1_b, W1)
    xw1 = _affine_mm(jnp.pad(x, ((0, 0), (0, 128 - 7))),
                     _pack_w(w1p, b1row, 128, 128), relu_in=False)
    conv1 = sparse_conv(xw1[:, :71], b1)

    r1 = jnp.maximum(conv1, 0.0)
    w2p, b2row = _fold_bn(r1.mean(0), r1.var(0), bn2_g, bn2_b, W2)
    xw2 = _affine_mm(jnp.pad(conv1, ((0, 0), (0, 128 - 71))),
                     _pack_w(w2p, b2row, 128, 256), relu_in=True)
    conv2 = sparse_conv(xw2[:, :135], b2)

    r2 = jnp.maximum(conv2, 0.0)
    w3p, b3row = _fold_bn(r2.mean(0), r2.var(0), bn3_g, bn3_b, W3)
    xw3 = _affine_mm(jnp.pad(conv2, ((0, 0), (0, 256 - 135))),
                     _pack_w(w3p, b3row, 256, 256), relu_in=True)
    conv3 = sparse_conv(xw3[:, :199], b3)

    h3 = jnp.pad(conv3, ((0, 0), (0, 256 - 199)))
    w1_aug = _pack_w(lin1_W, lin1_b, 256, 128)
    w2_aug = _pack_w(lin2_W, lin2_b, 128, 128)
    out = _pool_head(h3, batch.astype(jnp.int32), w1_aug, w2_aug, g)
    return out[:, :2]


# speedup vs baseline: 2.0379x; 1.7719x over previous
import functools

import jax
import jax.numpy as jnp
from jax.experimental import pallas as pl
from jax.experimental.pallas import tpu as pltpu

_BR = 1024


def _mm_body(h_ref, w_ref, o_ref, *, relu_in, ones_col):
    hb = h_ref[...]
    if relu_in:
        hb = jnp.maximum(hb, 0.0)
    col = jax.lax.broadcasted_iota(jnp.int32, hb.shape, 1)
    hb = jnp.where(col == ones_col, 1.0, hb)
    o_ref[...] = jnp.dot(hb, w_ref[...], preferred_element_type=jnp.float32)


def _affine_mm(h, w_aug, relu_in):
    n, finp = h.shape
    foutp = w_aug.shape[1]
    nb = -(-n // _BR)
    npad = nb * _BR
    if npad != n:
        h = jnp.pad(h, ((0, npad - n), (0, 0)))
    body = functools.partial(_mm_body, relu_in=relu_in, ones_col=finp - 1)
    out = pl.pallas_call(
        body,
        grid=(nb,),
        in_specs=[
            pl.BlockSpec((_BR, finp), lambda i: (i, 0)),
            pl.BlockSpec((finp, foutp), lambda i: (0, 0)),
        ],
        out_specs=pl.BlockSpec((_BR, foutp), lambda i: (i, 0)),
        out_shape=jax.ShapeDtypeStruct((npad, foutp), jnp.float32),
    )(h, w_aug)
    return out[:n]


def _pool_head_body(h_ref, bat_ref, w1_ref, w2_ref, o_ref, acc_ref, *, nb, g):
    i = pl.program_id(0)

    @pl.when(i == 0)
    def _init():
        acc_ref[...] = jnp.zeros_like(acc_ref)

    hb = jnp.maximum(h_ref[...], 0.0)
    col = jax.lax.broadcasted_iota(jnp.int32, hb.shape, 1)
    fp = hb.shape[1]
    hb = jnp.where(col == fp - 1, 1.0, hb)
    seg = bat_ref[0]
    oh = (jax.lax.broadcasted_iota(jnp.int32, (g, seg.shape[1]), 0) == seg)
    acc_ref[...] += jax.lax.dot(
        oh.astype(jnp.float32), hb, preferred_element_type=jnp.float32
    )

    @pl.when(i == nb - 1)
    def _finish():
        acc = acc_ref[...]
        cnt = acc[:, fp - 1 :]
        pooled = acc / jnp.maximum(cnt, 1.0)
        pcol = jax.lax.broadcasted_iota(jnp.int32, pooled.shape, 1)
        pooled = jnp.where(pcol == fp - 1, 1.0, pooled)
        t = jax.lax.dot(pooled, w1_ref[...], preferred_element_type=jnp.float32)
        tcol = jax.lax.broadcasted_iota(jnp.int32, t.shape, 1)
        t = jnp.where(tcol == t.shape[1] - 1, 1.0, t)
        o_ref[...] = jax.lax.dot(
            t, w2_ref[...], preferred_element_type=jnp.float32
        )


def _pool_head(h, batch, w1_aug, w2_aug, g):
    n, fp = h.shape
    nb = -(-n // _BR)
    npad = nb * _BR
    if npad != n:
        h = jnp.pad(h, ((0, npad - n), (0, 0)))
        batch = jnp.pad(batch, (0, npad - n), constant_values=2**30)
    bat3 = batch.reshape(nb, 1, _BR)
    body = functools.partial(_pool_head_body, nb=nb, g=g)
    out = pl.pallas_call(
        body,
        grid=(nb,),
        in_specs=[
            pl.BlockSpec((_BR, fp), lambda i: (i, 0)),
            pl.BlockSpec((1, 1, _BR), lambda i: (i, 0, 0)),
            pl.BlockSpec(w1_aug.shape, lambda i: (0, 0)),
            pl.BlockSpec(w2_aug.shape, lambda i: (0, 0)),
        ],
        out_specs=pl.BlockSpec((g, w2_aug.shape[1]), lambda i: (0, 0)),
        out_shape=jax.ShapeDtypeStruct((g, w2_aug.shape[1]), jnp.float32),
        scratch_shapes=[pltpu.VMEM((g, fp), jnp.float32)],
    )(h, bat3, w1_aug, w2_aug)
    return out


def _fold_bn(mean, var, gamma, beta, w):
    scale = gamma * jax.lax.rsqrt(var + 1e-5)
    bias = beta - mean * scale
    return scale[:, None] * w, bias @ w


def _pack_w(wp, brow, finp, foutp):
    fin, fout = wp.shape
    out = jnp.zeros((finp, foutp), jnp.float32)
    out = out.at[:fin, :fout].set(wp)
    out = out.at[finp - 1, :fout].set(brow)
    return out


def kernel(x, edge_index, batch, bn1_g, bn1_b, W1, b1, bn2_g, bn2_b, W2, b2,
           bn3_g, bn3_b, W3, b3, lin1_W, lin1_b, lin2_W, lin2_b):
    n = x.shape[0]
    g = 128
    src = edge_index[0]
    dst = edge_index[1]

    deg = jnp.ones((n,), jnp.float32).at[dst].add(1.0)
    dis = jax.lax.rsqrt(deg)[:, None]

    def sparse_conv(xw, b):
        y = xw * dis
        agg = jnp.zeros_like(xw).at[dst].add(y[src])
        return agg * dis + y * dis + b

    w1p, b1row = _fold_bn(x.mean(0), x.var(0), bn1_g, bn1_b, W1)
    xw1 = _affine_mm(jnp.pad(x, ((0, 0), (0, 128 - 7))),
                     _pack_w(w1p, b1row, 128, 128), relu_in=False)
    conv1 = sparse_conv(xw1[:, :71], b1)

    r1 = jnp.maximum(conv1, 0.0)
    w2p, b2row = _fold_bn(r1.mean(0), r1.var(0), bn2_g, bn2_b, W2)
    xw2 = _affine_mm(jnp.pad(conv1, ((0, 0), (0, 128 - 71))),
                     _pack_w(w2p, b2row, 128, 256), relu_in=True)
    conv2 = sparse_conv(xw2[:, :135], b2)

    r2 = jnp.maximum(conv2, 0.0)
    w3p, b3row = _fold_bn(r2.mean(0), r2.var(0), bn3_g, bn3_b, W3)
    xw3 = _affine_mm(jnp.pad(conv2, ((0, 0), (0, 256 - 135))),
                     _pack_w(w3p, b3row, 256, 256), relu_in=True)
    conv3 = sparse_conv(xw3[:, :199], b3)

    h3 = jnp.pad(conv3, ((0, 0), (0, 256 - 199)))
    w1_aug = _pack_w(lin1_W, lin1_b, 256, 128)
    w2_aug = _pack_w(lin2_W, lin2_b, 128, 128)
    out = _pool_head(h3, batch.astype(jnp.int32), w1_aug, w2_aug, g)
    return out[:, :2]


# speedup vs baseline: 2.0480x; 1.0049x over previous
import functools

import jax
import jax.numpy as jnp
from jax.experimental import pallas as pl
from jax.experimental.pallas import tpu as pltpu

_BR = 1024


def _mm_body(h_ref, w_ref, o_ref, *, relu_in, ones_col):
    hb = h_ref[...]
    if relu_in:
        hb = jnp.maximum(hb, 0.0)
    col = jax.lax.broadcasted_iota(jnp.int32, hb.shape, 1)
    hb = jnp.where(col == ones_col, 1.0, hb)
    o_ref[...] = jnp.dot(hb, w_ref[...], preferred_element_type=jnp.float32)


def _affine_mm(h, w_aug, relu_in):
    n, finp = h.shape
    foutp = w_aug.shape[1]
    nb = -(-n // _BR)
    npad = nb * _BR
    if npad != n:
        h = jnp.pad(h, ((0, npad - n), (0, 0)))
    body = functools.partial(_mm_body, relu_in=relu_in, ones_col=finp - 1)
    out = pl.pallas_call(
        body,
        grid=(nb,),
        in_specs=[
            pl.BlockSpec((_BR, finp), lambda i: (i, 0)),
            pl.BlockSpec((finp, foutp), lambda i: (0, 0)),
        ],
        out_specs=pl.BlockSpec((_BR, foutp), lambda i: (i, 0)),
        out_shape=jax.ShapeDtypeStruct((npad, foutp), jnp.float32),
    )(h, w_aug)
    return out[:n]


def _pool_head_body(h_ref, bat_ref, w1_ref, w2_ref, o_ref, acc_ref, *, nb, g):
    i = pl.program_id(0)

    @pl.when(i == 0)
    def _init():
        acc_ref[...] = jnp.zeros_like(acc_ref)

    hb = jnp.maximum(h_ref[...], 0.0)
    col = jax.lax.broadcasted_iota(jnp.int32, hb.shape, 1)
    fp = hb.shape[1]
    hb = jnp.where(col == fp - 1, 1.0, hb)
    seg = bat_ref[0]
    oh = (jax.lax.broadcasted_iota(jnp.int32, (g, seg.shape[1]), 0) == seg)
    acc_ref[...] += jax.lax.dot(
        oh.astype(jnp.float32), hb, preferred_element_type=jnp.float32
    )

    @pl.when(i == nb - 1)
    def _finish():
        acc = acc_ref[...]
        cnt = acc[:, fp - 1 :]
        pooled = acc / jnp.maximum(cnt, 1.0)
        pcol = jax.lax.broadcasted_iota(jnp.int32, pooled.shape, 1)
        pooled = jnp.where(pcol == fp - 1, 1.0, pooled)
        t = jax.lax.dot(pooled, w1_ref[...], preferred_element_type=jnp.float32)
        tcol = jax.lax.broadcasted_iota(jnp.int32, t.shape, 1)
        t = jnp.where(tcol == t.shape[1] - 1, 1.0, t)
        o_ref[...] = jax.lax.dot(
            t, w2_ref[...], preferred_element_type=jnp.float32
        )


def _pool_head(h, batch, w1_aug, w2_aug, g):
    n, fp = h.shape
    nb = -(-n // _BR)
    npad = nb * _BR
    if npad != n:
        h = jnp.pad(h, ((0, npad - n), (0, 0)))
        batch = jnp.pad(batch, (0, npad - n), constant_values=2**30)
    bat3 = batch.reshape(nb, 1, _BR)
    body = functools.partial(_pool_head_body, nb=nb, g=g)
    out = pl.pallas_call(
        body,
        grid=(nb,),
        in_specs=[
            pl.BlockSpec((_BR, fp), lambda i: (i, 0)),
            pl.BlockSpec((1, 1, _BR), lambda i: (i, 0, 0)),
            pl.BlockSpec(w1_aug.shape, lambda i: (0, 0)),
            pl.BlockSpec(w2_aug.shape, lambda i: (0, 0)),
        ],
        out_specs=pl.BlockSpec((g, w2_aug.shape[1]), lambda i: (0, 0)),
        out_shape=jax.ShapeDtypeStruct((g, w2_aug.shape[1]), jnp.float32),
        scratch_shapes=[pltpu.VMEM((g, fp), jnp.float32)],
    )(h, bat3, w1_aug, w2_aug)
    return out


def _fold_bn(mean, var, gamma, beta, w):
    scale = gamma * jax.lax.rsqrt(var + 1e-5)
    bias = beta - mean * scale
    return scale[:, None] * w, bias @ w


def _pack_w(wp, brow, finp, foutp):
    fin, fout = wp.shape
    out = jnp.zeros((finp, foutp), jnp.float32)
    out = out.at[:fin, :fout].set(wp)
    out = out.at[finp - 1, :fout].set(brow)
    return out


def kernel(x, edge_index, batch, bn1_g, bn1_b, W1, b1, bn2_g, bn2_b, W2, b2,
           bn3_g, bn3_b, W3, b3, lin1_W, lin1_b, lin2_W, lin2_b):
    n = x.shape[0]
    g = 128
    src = edge_index[0]
    dst = edge_index[1]

    deg = jnp.ones((n,), jnp.float32).at[dst].add(1.0)
    dis = jax.lax.rsqrt(deg)[:, None]

    def sparse_conv(xw, b):
        y = xw * dis
        agg = jnp.zeros_like(xw).at[dst].add(y[src])
        return (agg + y) * dis + b

    w1p, b1row = _fold_bn(x.mean(0), x.var(0), bn1_g, bn1_b, W1)
    xw1 = _affine_mm(jnp.pad(x, ((0, 0), (0, 128 - 7))),
                     _pack_w(w1p, b1row, 128, 128), relu_in=False)
    conv1 = sparse_conv(xw1[:, :71], b1)

    r1 = jnp.maximum(conv1, 0.0)
    w2p, b2row = _fold_bn(r1.mean(0), r1.var(0), bn2_g, bn2_b, W2)
    xw2 = _affine_mm(jnp.pad(conv1, ((0, 0), (0, 128 - 71))),
                     _pack_w(w2p, b2row, 128, 256), relu_in=True)
    conv2 = sparse_conv(xw2[:, :135], b2)

    r2 = jnp.maximum(conv2, 0.0)
    w3p, b3row = _fold_bn(r2.mean(0), r2.var(0), bn3_g, bn3_b, W3)
    xw3 = _affine_mm(jnp.pad(conv2, ((0, 0), (0, 256 - 135))),
                     _pack_w(w3p, b3row, 256, 256), relu_in=True)
    conv3 = sparse_conv(xw3[:, :199], b3)

    h3 = jnp.pad(conv3, ((0, 0), (0, 256 - 199)))
    w1_aug = _pack_w(lin1_W, lin1_b, 256, 128)
    w2_aug = _pack_w(lin2_W, lin2_b, 128, 128)
    out = _pool_head(h3, batch.astype(jnp.int32), w1_aug, w2_aug, g)
    return out[:, :2]
